# baseline (speedup 1.0000x reference)
import jax
import jax.numpy as jnp
from jax import lax
from jax.experimental import pallas as pl
from jax.experimental.pallas import tpu as pltpu
from jax.experimental.pallas import tpu_sc as plsc

N = 24
F = 128
E = 384
HID = 8
L = 16


def _sc_counts_body(ei_hbm, out_hbm, ei_v, idx0, idx1, idx2,
                    ones_v, zeros_v, c_sh):
    sid = lax.axis_index("s")

    @pl.when(sid == 0)
    def _():
        pltpu.sync_copy(ei_hbm, ei_v)
        for i in range(N * N // L):
            zeros_v[pl.ds(i * L, L)] = jnp.zeros((L,), jnp.float32)
        for i in range(128 // L):
            ones_v[pl.ds(i * L, L)] = jnp.ones((L,), jnp.float32)
        idx_refs = (idx0, idx1, idx2)
        for j in range(3):
            for i in range(128 // L):
                off = j * 128 + i * L
                sv = ei_v[0, pl.ds(off, L)]
                dv = ei_v[1, pl.ds(off, L)]
                idx_refs[j][pl.ds(i * L, L)] = dv * N + sv
        pltpu.sync_copy(zeros_v, c_sh)
        for j in range(3):
            pltpu.sync_copy(ones_v, c_sh.at[idx_refs[j]], add=True)
        pltpu.sync_copy(c_sh, out_hbm)


def _sc_counts(edge_index):
    mesh = plsc.VectorSubcoreMesh(core_axis_name="c", subcore_axis_name="s",
                                  num_cores=1)
    return pl.kernel(
        _sc_counts_body,
        out_type=jax.ShapeDtypeStruct((N * N,), jnp.float32),
        mesh=mesh,
        scratch_types=[
            pltpu.VMEM((2, E), jnp.int32),
            pltpu.VMEM((128,), jnp.int32),
            pltpu.VMEM((128,), jnp.int32),
            pltpu.VMEM((128,), jnp.int32),
            pltpu.VMEM((128,), jnp.float32),
            pltpu.VMEM((N * N,), jnp.float32),
            pltpu.VMEM_SHARED((N * N,), jnp.float32),
        ],
    )(edge_index)


def _mm(a, b):
    return jnp.dot(a, b, preferred_element_type=jnp.float32,
                   precision=lax.Precision.HIGHEST)


def _elu(v):
    return jnp.where(v > 0, v, jnp.exp(v) - 1.0)


def _tc_body(c_hbm, x_ref, w1_ref, b1_ref, w2_ref, b2_ref, f1_ref, f1b_ref,
             f2_ref, f2b_ref, out_ref, c_v, sem):
    f32 = jnp.float32
    NN = N * N

    pltpu.async_copy(c_hbm, c_v, sem).wait()

    c_flat = jnp.broadcast_to(c_v[:], (1, NN))
    uj = lax.broadcasted_iota(jnp.int32, (N, NN), 1)
    ud = lax.broadcasted_iota(jnp.int32, (N, NN), 0)
    U = ((uj // N) == ud).astype(f32) * jnp.broadcast_to(c_flat, (N, NN))
    vj = lax.broadcasted_iota(jnp.int32, (NN, N), 0)
    vs = lax.broadcasted_iota(jnp.int32, (NN, N), 1)
    V = ((vj % N) == vs).astype(f32)
    C = _mm(U, V)

    deg = jnp.sum(C, axis=0, keepdims=True)
    dinv = jnp.where(deg > 0, lax.rsqrt(jnp.where(deg > 0, deg, 1.0)), 0.0)
    S = -(C * jnp.transpose(dinv, (1, 0))) * dinv

    x = x_ref[:, :]
    P0 = _mm(x, w1_ref[0])
    P1 = _mm(x, w1_ref[1])
    P2 = _mm(x, w1_ref[2])
    b1 = jnp.broadcast_to(b1_ref[:], (N, HID))
    h = _elu(P0 + _mm(S, P1) + 2.0 * _mm(S, _mm(S, P2)) - P2 + b1)

    Q0 = _mm(h, w2_ref[0])
    Q1 = _mm(h, w2_ref[1])
    Q2 = _mm(h, w2_ref[2])
    b2 = jnp.broadcast_to(b2_ref[:], (N, HID))
    g = _elu(Q0 + _mm(S, Q1) + 2.0 * _mm(S, _mm(S, Q2)) - Q2 + b2)

    NH = N * HID
    rk = lax.broadcasted_iota(jnp.int32, (NH, N), 0)
    rn = lax.broadcasted_iota(jnp.int32, (NH, N), 1)
    Rep = ((rk // HID) == rn).astype(f32)
    Gr = _mm(Rep, g)
    fk = lax.broadcasted_iota(jnp.int32, (NH, HID), 0)
    ff = lax.broadcasted_iota(jnp.int32, (NH, HID), 1)
    sel = ((fk % HID) == ff).astype(f32)
    gcol = jnp.sum(Gr * sel, axis=1, keepdims=True)

    f1T = jnp.transpose(f1_ref[:, :], (1, 0))
    f2T = jnp.transpose(f2_ref[:, :], (1, 0))
    f1b = jnp.transpose(jnp.broadcast_to(f1b_ref[:], (1, 64)), (1, 0))
    f2b = jnp.transpose(jnp.broadcast_to(f2b_ref[:], (1, 2)), (1, 0))
    v1 = _mm(f1T, gcol) + f1b
    v2 = _mm(f2T, v1) + f2b

    m = jnp.max(v2)
    lse = m + jnp.log(jnp.sum(jnp.exp(v2 - m)))
    out_ref[:, :] = jnp.transpose(v2 - lse, (1, 0))


def _tc_tail(c_flat, x, W1, b1, W2, b2, fc1_w, fc1_b, fc2_w, fc2_b):
    return pl.pallas_call(
        _tc_body,
        out_shape=jax.ShapeDtypeStruct((1, 2), jnp.float32),
        in_specs=[pl.BlockSpec(memory_space=pl.ANY)]
        + [pl.BlockSpec(memory_space=pltpu.VMEM)] * 9,
        out_specs=pl.BlockSpec(memory_space=pltpu.VMEM),
        scratch_shapes=[pltpu.VMEM((N * N,), jnp.float32),
                        pltpu.SemaphoreType.DMA],
    )(c_flat, x, W1, b1, W2, b2, fc1_w, fc1_b, fc2_w, fc2_b)


@jax.jit
def _run(x, edge_index, W1, b1, W2, b2, fc1_w, fc1_b, fc2_w, fc2_b):
    c_flat = _sc_counts(edge_index)
    return _tc_tail(c_flat, x, W1, b1, W2, b2, fc1_w, fc1_b, fc2_w, fc2_b)


def kernel(x, edge_index, W1, b1, W2, b2, fc1_w, fc1_b, fc2_w, fc2_b):
    return _run(x, edge_index, W1, b1, W2, b2, fc1_w, fc1_b, fc2_w, fc2_b)


# speedup vs baseline: 1.0331x; 1.0331x over previous
import jax
import jax.numpy as jnp
from jax import lax
from jax.experimental import pallas as pl
from jax.experimental.pallas import tpu as pltpu
from jax.experimental.pallas import tpu_sc as plsc

N = 24
F = 128
E = 384
HID = 8
L = 16


def _sc_counts_body(ei_hbm, out_hbm, ei_v, idx0, idx1, idx2,
                    ones_v, zeros_v, c_sh, sem):
    sid = lax.axis_index("s")

    @pl.when(sid == 0)
    def _():
        pltpu.sync_copy(ei_hbm, ei_v)
        for i in range(N * N // L):
            zeros_v[pl.ds(i * L, L)] = jnp.zeros((L,), jnp.float32)
        for i in range(128 // L):
            ones_v[pl.ds(i * L, L)] = jnp.ones((L,), jnp.float32)
        idx_refs = (idx0, idx1, idx2)
        for j in range(3):
            for i in range(128 // L):
                off = j * 128 + i * L
                sv = ei_v[0, pl.ds(off, L)]
                dv = ei_v[1, pl.ds(off, L)]
                idx_refs[j][pl.ds(i * L, L)] = dv * N + sv
        pltpu.sync_copy(zeros_v, c_sh)
        cps = [pltpu.async_copy(ones_v, c_sh.at[idx_refs[j]], sem, add=True)
               for j in range(3)]
        for cp in cps:
            cp.wait()
        pltpu.sync_copy(c_sh, out_hbm)


def _sc_counts(edge_index):
    mesh = plsc.VectorSubcoreMesh(core_axis_name="c", subcore_axis_name="s",
                                  num_cores=1)
    return pl.kernel(
        _sc_counts_body,
        out_type=jax.ShapeDtypeStruct((N * N,), jnp.float32),
        mesh=mesh,
        scratch_types=[
            pltpu.VMEM((2, E), jnp.int32),
            pltpu.VMEM((128,), jnp.int32),
            pltpu.VMEM((128,), jnp.int32),
            pltpu.VMEM((128,), jnp.int32),
            pltpu.VMEM((128,), jnp.float32),
            pltpu.VMEM((N * N,), jnp.float32),
            pltpu.VMEM_SHARED((N * N,), jnp.float32),
            pltpu.SemaphoreType.DMA,
        ],
    )(edge_index)


def _mm(a, b):
    return jnp.dot(a, b, preferred_element_type=jnp.float32,
                   precision=lax.Precision.HIGHEST)


def _elu(v):
    return jnp.where(v > 0, v, jnp.exp(v) - 1.0)


def _tc_body(c_ref, x_ref, w1_ref, b1_ref, w2_ref, b2_ref, f1_ref, f1b_ref,
             f2_ref, f2b_ref, out_ref):
    f32 = jnp.float32
    NN = N * N

    c_flat = jnp.broadcast_to(c_ref[:], (1, NN))
    uj = lax.broadcasted_iota(jnp.int32, (N, NN), 1)
    ud = lax.broadcasted_iota(jnp.int32, (N, NN), 0)
    U = ((uj // N) == ud).astype(f32) * jnp.broadcast_to(c_flat, (N, NN))
    vj = lax.broadcasted_iota(jnp.int32, (NN, N), 0)
    vs = lax.broadcasted_iota(jnp.int32, (NN, N), 1)
    V = ((vj % N) == vs).astype(f32)
    C = _mm(U, V)

    deg = jnp.sum(C, axis=0, keepdims=True)
    dinv = jnp.where(deg > 0, lax.rsqrt(jnp.where(deg > 0, deg, 1.0)), 0.0)
    S = -(C * jnp.transpose(dinv, (1, 0))) * dinv

    x = x_ref[:, :]
    P0 = _mm(x, w1_ref[0])
    P1 = _mm(x, w1_ref[1])
    P2 = _mm(x, w1_ref[2])
    b1 = jnp.broadcast_to(b1_ref[:], (N, HID))
    h = _elu(P0 + _mm(S, P1) + 2.0 * _mm(S, _mm(S, P2)) - P2 + b1)

    Q0 = _mm(h, w2_ref[0])
    Q1 = _mm(h, w2_ref[1])
    Q2 = _mm(h, w2_ref[2])
    b2 = jnp.broadcast_to(b2_ref[:], (N, HID))
    g = _elu(Q0 + _mm(S, Q1) + 2.0 * _mm(S, _mm(S, Q2)) - Q2 + b2)

    NH = N * HID
    rk = lax.broadcasted_iota(jnp.int32, (NH, N), 0)
    rn = lax.broadcasted_iota(jnp.int32, (NH, N), 1)
    Rep = ((rk // HID) == rn).astype(f32)
    Gr = _mm(Rep, g)
    fk = lax.broadcasted_iota(jnp.int32, (NH, HID), 0)
    ff = lax.broadcasted_iota(jnp.int32, (NH, HID), 1)
    sel = ((fk % HID) == ff).astype(f32)
    gcol = jnp.sum(Gr * sel, axis=1, keepdims=True)

    f1T = jnp.transpose(f1_ref[:, :], (1, 0))
    f2T = jnp.transpose(f2_ref[:, :], (1, 0))
    f1b = jnp.transpose(jnp.broadcast_to(f1b_ref[:], (1, 64)), (1, 0))
    f2b = jnp.transpose(jnp.broadcast_to(f2b_ref[:], (1, 2)), (1, 0))
    v1 = _mm(f1T, gcol) + f1b
    v2 = _mm(f2T, v1) + f2b

    m = jnp.max(v2)
    lse = m + jnp.log(jnp.sum(jnp.exp(v2 - m)))
    out_ref[:, :] = jnp.transpose(v2 - lse, (1, 0))


def _tc_tail(c_flat, x, W1, b1, W2, b2, fc1_w, fc1_b, fc2_w, fc2_b):
    return pl.pallas_call(
        _tc_body,
        out_shape=jax.ShapeDtypeStruct((1, 2), jnp.float32),
    )(c_flat, x, W1, b1, W2, b2, fc1_w, fc1_b, fc2_w, fc2_b)


@jax.jit
def _run(x, edge_index, W1, b1, W2, b2, fc1_w, fc1_b, fc2_w, fc2_b):
    c_flat = _sc_counts(edge_index)
    return _tc_tail(c_flat, x, W1, b1, W2, b2, fc1_w, fc1_b, fc2_w, fc2_b)


def kernel(x, edge_index, W1, b1, W2, b2, fc1_w, fc1_b, fc2_w, fc2_b):
    return _run(x, edge_index, W1, b1, W2, b2, fc1_w, fc1_b, fc2_w, fc2_b)


# speedup vs baseline: 1.0428x; 1.0094x over previous
import jax
import jax.numpy as jnp
from jax import lax
from jax.experimental import pallas as pl
from jax.experimental.pallas import tpu as pltpu
from jax.experimental.pallas import tpu_sc as plsc

N = 24
F = 128
E = 384
HID = 8
L = 16


def _sc_counts_body(ei_hbm, out_hbm, ei_v, idx0, idx1, idx2,
                    ones_v, zeros_v, c_sh, sem):
    sid = lax.axis_index("s")

    @pl.when(sid == 0)
    def _():
        pltpu.sync_copy(ei_hbm, ei_v)
        for i in range(N * N // L):
            zeros_v[pl.ds(i * L, L)] = jnp.zeros((L,), jnp.float32)
        for i in range(128 // L):
            ones_v[pl.ds(i * L, L)] = jnp.ones((L,), jnp.float32)
        idx_refs = (idx0, idx1, idx2)
        for j in range(3):
            for i in range(128 // L):
                off = j * 128 + i * L
                sv = ei_v[0, pl.ds(off, L)]
                dv = ei_v[1, pl.ds(off, L)]
                idx_refs[j][pl.ds(i * L, L)] = dv * N + sv
        pltpu.sync_copy(zeros_v, c_sh)
        cps = [pltpu.async_copy(ones_v, c_sh.at[idx_refs[j]], sem, add=True)
               for j in range(3)]
        for cp in cps:
            cp.wait()
        pltpu.sync_copy(c_sh, out_hbm)


def _sc_counts(edge_index):
    mesh = plsc.VectorSubcoreMesh(core_axis_name="c", subcore_axis_name="s",
                                  num_cores=1)
    return pl.kernel(
        _sc_counts_body,
        out_type=jax.ShapeDtypeStruct((N * N,), jnp.float32),
        mesh=mesh,
        scratch_types=[
            pltpu.VMEM((2, E), jnp.int32),
            pltpu.VMEM((128,), jnp.int32),
            pltpu.VMEM((128,), jnp.int32),
            pltpu.VMEM((128,), jnp.int32),
            pltpu.VMEM((128,), jnp.float32),
            pltpu.VMEM((N * N,), jnp.float32),
            pltpu.VMEM_SHARED((N * N,), jnp.float32),
            pltpu.SemaphoreType.DMA,
        ],
    )(edge_index)


def _mm(a, b):
    return jnp.dot(a, b, preferred_element_type=jnp.float32,
                   precision=lax.Precision.HIGHEST)


def _elu(v):
    return jnp.where(v > 0, v, jnp.exp(v) - 1.0)


def _tc_pre_body(x_ref, w1_ref, f1_ref, f1b_ref, f2_ref, f2b_ref,
                 p0_ref, p1_ref, p2_ref, wc2_ref, bc_ref):
    x = x_ref[:, :]
    p0_ref[:, :] = _mm(x, w1_ref[0])
    p1_ref[:, :] = _mm(x, w1_ref[1])
    p2_ref[:, :] = _mm(x, w1_ref[2])
    f1T = jnp.transpose(f1_ref[:, :], (1, 0))
    f2T = jnp.transpose(f2_ref[:, :], (1, 0))

    def _col(row_ref, n):
        i0 = lax.broadcasted_iota(jnp.int32, (n, n), 0)
        i1 = lax.broadcasted_iota(jnp.int32, (n, n), 1)
        m = (i0 == i1).astype(jnp.float32)
        return jnp.sum(m * jnp.broadcast_to(row_ref[:], (n, n)),
                       axis=1, keepdims=True)

    f1b = _col(f1b_ref, 64)
    f2b = _col(f2b_ref, 2)
    wc2_ref[:, :] = _mm(f2T, f1T)
    bc_ref[:, :] = _mm(f2T, f1b) + f2b


def _tc_pre(x, W1, fc1_w, fc1_b, fc2_w, fc2_b):
    NH = N * HID
    return pl.pallas_call(
        _tc_pre_body,
        out_shape=(
            jax.ShapeDtypeStruct((N, HID), jnp.float32),
            jax.ShapeDtypeStruct((N, HID), jnp.float32),
            jax.ShapeDtypeStruct((N, HID), jnp.float32),
            jax.ShapeDtypeStruct((2, NH), jnp.float32),
            jax.ShapeDtypeStruct((2, 1), jnp.float32),
        ),
    )(x, W1, fc1_w, fc1_b, fc2_w, fc2_b)


def _tc_body(c_ref, p0_ref, p1_ref, p2_ref, b1_ref, w2_ref, b2_ref,
             wc2_ref, bc_ref, out_ref):
    f32 = jnp.float32
    NN = N * N

    c_flat = jnp.broadcast_to(c_ref[:], (1, NN))
    uj = lax.broadcasted_iota(jnp.int32, (N, NN), 1)
    ud = lax.broadcasted_iota(jnp.int32, (N, NN), 0)
    U = ((uj // N) == ud).astype(f32) * jnp.broadcast_to(c_flat, (N, NN))
    vj = lax.broadcasted_iota(jnp.int32, (NN, N), 0)
    vs = lax.broadcasted_iota(jnp.int32, (NN, N), 1)
    V = ((vj % N) == vs).astype(f32)
    C = _mm(U, V)

    deg = jnp.sum(C, axis=0, keepdims=True)
    dinv = jnp.where(deg > 0, lax.rsqrt(jnp.where(deg > 0, deg, 1.0)), 0.0)
    S = -(C * jnp.transpose(dinv, (1, 0))) * dinv

    P0 = p0_ref[:, :]
    P1 = p1_ref[:, :]
    P2 = p2_ref[:, :]
    b1 = jnp.broadcast_to(b1_ref[:], (N, HID))
    h = _elu(P0 + _mm(S, P1) + 2.0 * _mm(S, _mm(S, P2)) - P2 + b1)

    Q0 = _mm(h, w2_ref[0])
    Q1 = _mm(h, w2_ref[1])
    Q2 = _mm(h, w2_ref[2])
    b2 = jnp.broadcast_to(b2_ref[:], (N, HID))
    g = _elu(Q0 + _mm(S, Q1) + 2.0 * _mm(S, _mm(S, Q2)) - Q2 + b2)

    NH = N * HID
    rk = lax.broadcasted_iota(jnp.int32, (NH, N), 0)
    rn = lax.broadcasted_iota(jnp.int32, (NH, N), 1)
    Rep = ((rk // HID) == rn).astype(f32)
    Gr = _mm(Rep, g)
    fk = lax.broadcasted_iota(jnp.int32, (NH, HID), 0)
    ff = lax.broadcasted_iota(jnp.int32, (NH, HID), 1)
    sel = ((fk % HID) == ff).astype(f32)
    gcol = jnp.sum(Gr * sel, axis=1, keepdims=True)

    v2 = _mm(wc2_ref[:, :], gcol) + bc_ref[:, :]

    m = jnp.max(v2)
    lse = m + jnp.log(jnp.sum(jnp.exp(v2 - m)))
    out_ref[:, :] = jnp.transpose(v2 - lse, (1, 0))


def _tc_tail(c_flat, p0, p1, p2, b1, W2, b2, wc2, bc):
    return pl.pallas_call(
        _tc_body,
        out_shape=jax.ShapeDtypeStruct((1, 2), jnp.float32),
    )(c_flat, p0, p1, p2, b1, W2, b2, wc2, bc)


@jax.jit
def _run(x, edge_index, W1, b1, W2, b2, fc1_w, fc1_b, fc2_w, fc2_b):
    c_flat = _sc_counts(edge_index)
    p0, p1, p2, wc2, bc = _tc_pre(x, W1, fc1_w, fc1_b, fc2_w, fc2_b)
    return _tc_tail(c_flat, p0, p1, p2, b1, W2, b2, wc2, bc)


def kernel(x, edge_index, W1, b1, W2, b2, fc1_w, fc1_b, fc2_w, fc2_b):
    return _run(x, edge_index, W1, b1, W2, b2, fc1_w, fc1_b, fc2_w, fc2_b)
